# Initial kernel scaffold; baseline (speedup 1.0000x reference)
import jax
import jax.numpy as jnp
from jax.experimental import pallas as pl


def kernel(trg_token_ids_batch):
    raise NotImplementedError("write your pallas kernel here")


# speedup vs baseline: 1.8838x; 1.8838x over previous
import jax
import jax.numpy as jnp
from jax.experimental import pallas as pl

_V = 100000
_B = 1024
_EPS = 0.1 / (_V - 2)
_CONF = 0.9
_BR = 16


def _body(trg_ref, out_ref):
    tv = trg_ref[...]
    cols = jax.lax.broadcasted_iota(jnp.int32, (_BR, _V), 1)
    out = jnp.where(cols == tv, _CONF, _EPS).astype(jnp.float32)
    out = jnp.where((cols == 0) | (tv == 0), 0.0, out)
    out_ref[...] = out


def kernel(trg_token_ids_batch):
    return pl.pallas_call(
        _body,
        grid=(_B // _BR,),
        in_specs=[pl.BlockSpec((_BR, 1), lambda i: (i, 0))],
        out_specs=pl.BlockSpec((_BR, _V), lambda i: (i, 0)),
        out_shape=jax.ShapeDtypeStruct((_B, _V), jnp.float32),
    )(trg_token_ids_batch)
